# Initial kernel scaffold; baseline (speedup 1.0000x reference)
import jax
import jax.numpy as jnp
from jax.experimental import pallas as pl
from jax.experimental.pallas import tpu as pltpu

_RADIUS = 0.8
_B = 1024
_K = 16384
_BB = 32


def _vm_kernel(craw_ref, ct_ref, tpf_ref, out_ref, idx_ref, cnt_ref):
    craw = craw_ref[...]
    ct = ct_ref[...]
    tpf = tpf_ref[...]

    cn = jnp.sqrt(jnp.sum(craw * craw, axis=0, keepdims=True))
    cdir = craw / (cn + 1e-8)

    tdr = jnp.concatenate([tpf[:, 2:3], tpf[:, 6:7], tpf[:, 10:11]], axis=1)
    tn = jnp.sqrt(jnp.sum(tdr * tdr, axis=1, keepdims=True))
    tdir = tdr / (tn + 1e-8)
    torig = jnp.concatenate([tpf[:, 3:4], tpf[:, 7:8], tpf[:, 11:12]], axis=1)

    cos = jax.lax.dot_general(tdir, cdir, (((1,), (0,)), ((), ())),
                              preferred_element_type=jnp.float32)
    cos = jnp.clip(cos, -0.999999, 0.999999)
    t = jnp.arccos(cos)

    oc = jax.lax.dot_general(torig, ct, (((1,), (0,)), ((), ())),
                             preferred_element_type=jnp.float32)
    o2 = jnp.sum(torig * torig, axis=1, keepdims=True)
    c2 = jnp.sum(ct * ct, axis=0, keepdims=True)
    d = jnp.sqrt(jnp.maximum(o2 + c2 - 2.0 * oc, 0.0) + 1e-12)

    in_r = d <= _RADIUS
    any_in = jnp.max(in_r.astype(jnp.int32), axis=1, keepdims=True) > 0
    mask = jnp.logical_and(any_in, jnp.logical_not(in_r))
    t_m = jnp.where(mask, 10000.0, t)
    d_m = jnp.where(mask, 10000.0, d)

    tmin = jnp.min(t_m, axis=1, keepdims=True)
    iota = jax.lax.broadcasted_iota(jnp.int32, t_m.shape, 1)
    idx = jnp.min(jnp.where(t_m == tmin, iota, _K), axis=1, keepdims=True)
    dsel = jnp.sum(jnp.where(iota == idx, d_m, 0.0), axis=1, keepdims=True)
    cnt = jnp.sum(in_r.astype(jnp.int32), axis=1, keepdims=True)

    tr9 = jnp.concatenate([tpf[:, 0:3], tpf[:, 4:7], tpf[:, 8:11]], axis=1)
    out_ref[...] = jnp.concatenate([tmin, dsel, tr9], axis=1)
    idx_ref[...] = idx
    cnt_ref[...] = cnt


def kernel(candidate_rotations, candidate_translations, target_pose):
    craw3 = candidate_rotations[:, :, 2].T
    ct3 = candidate_translations.T
    tpf = target_pose.reshape(_B, 16)

    out_f, idx, cnt = pl.pallas_call(
        _vm_kernel,
        grid=(_B // _BB,),
        in_specs=[
            pl.BlockSpec((3, _K), lambda b: (0, 0)),
            pl.BlockSpec((3, _K), lambda b: (0, 0)),
            pl.BlockSpec((_BB, 16), lambda b: (b, 0)),
        ],
        out_specs=[
            pl.BlockSpec((_BB, 11), lambda b: (b, 0)),
            pl.BlockSpec((_BB, 1), lambda b: (b, 0)),
            pl.BlockSpec((_BB, 1), lambda b: (b, 0)),
        ],
        out_shape=[
            jax.ShapeDtypeStruct((_B, 11), jnp.float32),
            jax.ShapeDtypeStruct((_B, 1), jnp.int32),
            jax.ShapeDtypeStruct((_B, 1), jnp.int32),
        ],
        compiler_params=pltpu.CompilerParams(
            dimension_semantics=("parallel",),
        ),
    )(craw3, ct3, tpf)
    return out_f, idx[:, 0], cnt[:, 0]


# speedup vs baseline: 1.3874x; 1.3874x over previous
import jax
import jax.numpy as jnp
from jax.experimental import pallas as pl
from jax.experimental.pallas import tpu as pltpu

_RADIUS = 0.8
_B = 1024
_K = 16384
_BB = 32


def _vm_kernel(craw_ref, ct_ref, tpf_ref, out_ref, idx_ref, cnt_ref):
    craw = craw_ref[...]
    ct = ct_ref[...]
    tpf = tpf_ref[...]

    cn = jnp.sqrt(jnp.sum(craw * craw, axis=0, keepdims=True))
    cdir = craw / (cn + 1e-8)

    tdr = jnp.concatenate([tpf[:, 2:3], tpf[:, 6:7], tpf[:, 10:11]], axis=1)
    tn = jnp.sqrt(jnp.sum(tdr * tdr, axis=1, keepdims=True))
    tdir = tdr / (tn + 1e-8)
    torig = jnp.concatenate([tpf[:, 3:4], tpf[:, 7:8], tpf[:, 11:12]], axis=1)

    cos = jax.lax.dot_general(tdir, cdir, (((1,), (0,)), ((), ())),
                              preferred_element_type=jnp.float32)
    cos = jnp.clip(cos, -0.999999, 0.999999)
    t = 2.0 * jnp.arctan2(jnp.sqrt(1.0 - cos * cos), 1.0 + cos)

    oc = jax.lax.dot_general(torig, ct, (((1,), (0,)), ((), ())),
                             preferred_element_type=jnp.float32)
    o2 = jnp.sum(torig * torig, axis=1, keepdims=True)
    c2 = jnp.sum(ct * ct, axis=0, keepdims=True)
    d = jnp.sqrt(jnp.maximum(o2 + c2 - 2.0 * oc, 0.0) + 1e-12)

    in_r = d <= _RADIUS
    any_in = jnp.max(in_r.astype(jnp.int32), axis=1, keepdims=True) > 0
    mask = jnp.logical_and(any_in, jnp.logical_not(in_r))
    t_m = jnp.where(mask, 10000.0, t)
    d_m = jnp.where(mask, 10000.0, d)

    tmin = jnp.min(t_m, axis=1, keepdims=True)
    iota = jax.lax.broadcasted_iota(jnp.int32, t_m.shape, 1)
    idx = jnp.min(jnp.where(t_m == tmin, iota, _K), axis=1, keepdims=True)
    dsel = jnp.sum(jnp.where(iota == idx, d_m, 0.0), axis=1, keepdims=True)
    cnt = jnp.sum(in_r.astype(jnp.int32), axis=1, keepdims=True)

    tr9 = jnp.concatenate([tpf[:, 0:3], tpf[:, 4:7], tpf[:, 8:11]], axis=1)
    out_ref[...] = jnp.concatenate([tmin, dsel, tr9], axis=1)
    idx_ref[...] = idx
    cnt_ref[...] = cnt


def kernel(candidate_rotations, candidate_translations, target_pose):
    craw3 = candidate_rotations[:, :, 2].T
    ct3 = candidate_translations.T
    tpf = target_pose.reshape(_B, 16)

    out_f, idx, cnt = pl.pallas_call(
        _vm_kernel,
        grid=(_B // _BB,),
        in_specs=[
            pl.BlockSpec((3, _K), lambda b: (0, 0)),
            pl.BlockSpec((3, _K), lambda b: (0, 0)),
            pl.BlockSpec((_BB, 16), lambda b: (b, 0)),
        ],
        out_specs=[
            pl.BlockSpec((_BB, 11), lambda b: (b, 0)),
            pl.BlockSpec((_BB, 1), lambda b: (b, 0)),
            pl.BlockSpec((_BB, 1), lambda b: (b, 0)),
        ],
        out_shape=[
            jax.ShapeDtypeStruct((_B, 11), jnp.float32),
            jax.ShapeDtypeStruct((_B, 1), jnp.int32),
            jax.ShapeDtypeStruct((_B, 1), jnp.int32),
        ],
        compiler_params=pltpu.CompilerParams(
            dimension_semantics=("parallel",),
        ),
    )(craw3, ct3, tpf)
    return out_f, idx[:, 0], cnt[:, 0]


# speedup vs baseline: 3.4007x; 2.4512x over previous
import jax
import jax.numpy as jnp
from jax.experimental import pallas as pl
from jax.experimental.pallas import tpu as pltpu

_RADIUS = 0.8
_Q_THRESH = float.fromhex("0x1.47ae16p-1")
_B = 1024
_K = 16384
_BB = 32


def _vm_kernel(craw_ref, ct_ref, tpf_ref, out_ref, idx_ref, cnt_ref):
    craw = craw_ref[...]
    ct = ct_ref[...]
    tpf = tpf_ref[...]

    cn = jnp.sqrt(jnp.sum(craw * craw, axis=0, keepdims=True))
    cdir = craw / (cn + 1e-8)

    tdr = jnp.concatenate([tpf[:, 2:3], tpf[:, 6:7], tpf[:, 10:11]], axis=1)
    tn = jnp.sqrt(jnp.sum(tdr * tdr, axis=1, keepdims=True))
    tdir = tdr / (tn + 1e-8)
    torig = jnp.concatenate([tpf[:, 3:4], tpf[:, 7:8], tpf[:, 11:12]], axis=1)

    cos = jax.lax.dot_general(tdir, cdir, (((1,), (0,)), ((), ())),
                              preferred_element_type=jnp.float32)
    cos = jnp.clip(cos, -0.999999, 0.999999)

    oc = jax.lax.dot_general(torig, ct, (((1,), (0,)), ((), ())),
                             preferred_element_type=jnp.float32)
    o2 = jnp.sum(torig * torig, axis=1, keepdims=True)
    c2 = jnp.sum(ct * ct, axis=0, keepdims=True)
    qv = jnp.maximum(o2 + c2 - 2.0 * oc, 0.0) + 1e-12

    in_r = qv <= _Q_THRESH
    cnt = jnp.sum(in_r.astype(jnp.int32), axis=1, keepdims=True)
    any_in = cnt > 0

    eff = jnp.where(jnp.logical_and(any_in, jnp.logical_not(in_r)), -3.0, cos)

    m = jnp.max(eff, axis=1, keepdims=True)
    iota = jax.lax.broadcasted_iota(jnp.int32, eff.shape, 1)
    idx = jnp.min(jnp.where(eff == m, iota, _K), axis=1, keepdims=True)
    qsel = jnp.max(jnp.where(iota == idx, qv, 0.0), axis=1, keepdims=True)

    angle = 2.0 * jnp.arctan2(jnp.sqrt(1.0 - m * m), 1.0 + m)
    dist = jnp.sqrt(qsel)

    tr9 = jnp.concatenate([tpf[:, 0:3], tpf[:, 4:7], tpf[:, 8:11]], axis=1)
    out_ref[...] = jnp.concatenate([angle, dist, tr9], axis=1)
    idx_ref[...] = idx
    cnt_ref[...] = cnt


def kernel(candidate_rotations, candidate_translations, target_pose):
    craw3 = candidate_rotations[:, :, 2].T
    ct3 = candidate_translations.T
    tpf = target_pose.reshape(_B, 16)

    out_f, idx, cnt = pl.pallas_call(
        _vm_kernel,
        grid=(_B // _BB,),
        in_specs=[
            pl.BlockSpec((3, _K), lambda b: (0, 0)),
            pl.BlockSpec((3, _K), lambda b: (0, 0)),
            pl.BlockSpec((_BB, 16), lambda b: (b, 0)),
        ],
        out_specs=[
            pl.BlockSpec((_BB, 11), lambda b: (b, 0)),
            pl.BlockSpec((_BB, 1), lambda b: (b, 0)),
            pl.BlockSpec((_BB, 1), lambda b: (b, 0)),
        ],
        out_shape=[
            jax.ShapeDtypeStruct((_B, 11), jnp.float32),
            jax.ShapeDtypeStruct((_B, 1), jnp.int32),
            jax.ShapeDtypeStruct((_B, 1), jnp.int32),
        ],
        compiler_params=pltpu.CompilerParams(
            dimension_semantics=("parallel",),
        ),
    )(craw3, ct3, tpf)
    return out_f, idx[:, 0], cnt[:, 0]
